# Initial kernel scaffold; baseline (speedup 1.0000x reference)
import jax
import jax.numpy as jnp
from jax.experimental import pallas as pl


def kernel(z, edge_index, edge_attr, emb, W1, b1, W2, b2):
    raise NotImplementedError("write your pallas kernel here")


# speedup vs baseline: 1.7894x; 1.7894x over previous
import functools

import jax
import jax.numpy as jnp
from jax import lax
from jax.experimental import pallas as pl
from jax.experimental.pallas import tpu as pltpu
from jax.experimental.pallas import tpu_sc as plsc

N = 10000
E = 320000
D = 128
NUM_CONVS = 3

NC = 2
NS = 16
NW = NC * NS

NP = 10240
CHUNK = 128
EPT = 10112
NCHUNKS = EPT // CHUNK
E_PAD = NW * EPT
ROWS_PER_TILE = NP // NS
EMB_ROWS_PER_TILE = NP // NW
EMB_CHUNK = 80

@functools.cache
def _mesh():
    return plsc.VectorSubcoreMesh(
        core_axis_name="c", subcore_axis_name="s",
        num_cores=NC, num_subcores=NS)


def _relu_add_chunk(rows, ea, nrows):
    def body(r, carry):
        for j in range(D // 16):
            sl = pl.ds(j * 16, 16)
            v = rows[r, sl] + ea[r, sl]
            rows[r, sl] = jnp.maximum(v, 0.0)
        return carry
    lax.fori_loop(0, nrows, body, 0, unroll=2)


@functools.cache
def _embed_kernel():
    return pl.kernel(
        _embed_body,
        out_type=jax.ShapeDtypeStruct((NP, D), jnp.float32),
        mesh=_mesh(),
        scratch_types=[
            pltpu.VMEM((EMB_CHUNK,), jnp.int32),
            pltpu.VMEM((EMB_CHUNK, D), jnp.float32),
            pltpu.SemaphoreType.DMA,
        ],
    )


def _embed_body(emb_hbm, z_hbm, out_hbm, idx_v, rows_v, sem):
    cid = lax.axis_index("c")
    sid = lax.axis_index("s")
    wid = sid * NC + cid
    base = pl.multiple_of(wid * EMB_ROWS_PER_TILE, 8)
    for i in range(EMB_ROWS_PER_TILE // EMB_CHUNK):
        b = pl.multiple_of(base + i * EMB_CHUNK, 8)
        pltpu.sync_copy(z_hbm.at[pl.ds(b, EMB_CHUNK)], idx_v)
        pltpu.async_copy(emb_hbm.at[idx_v], rows_v, sem).wait()
        pltpu.sync_copy(rows_v, out_hbm.at[pl.ds(b, EMB_CHUNK)])


@functools.cache
def _conv_kernel():
    return pl.kernel(
        _conv_body,
        out_type=[jax.ShapeDtypeStruct((NP, D), jnp.float32),
                  jax.ShapeDtypeStruct((NP, D), jnp.float32)],
        mesh=_mesh(),
        scratch_types=[
            pltpu.VMEM((CHUNK,), jnp.int32),
            pltpu.VMEM((CHUNK,), jnp.int32),
            pltpu.VMEM((CHUNK, D), jnp.float32),
            pltpu.VMEM((CHUNK, D), jnp.float32),
            pltpu.VMEM_SHARED((NP, D), jnp.float32),
            pltpu.SemaphoreType.DMA,
        ],
    )


def _conv_body(x_hbm, src_hbm, dst_hbm, ea_hbm, zeros_hbm,
               out0_hbm, out1_hbm,
               idx_s, idx_d, rows, ea, acc, sem):
    cid = lax.axis_index("c")
    sid = lax.axis_index("s")
    wid = sid * NC + cid

    r0 = pl.multiple_of(sid * ROWS_PER_TILE, 8)
    for j in range(ROWS_PER_TILE // CHUNK):
        pltpu.sync_copy(zeros_hbm, acc.at[pl.ds(r0 + j * CHUNK, CHUNK)])
    plsc.subcore_barrier()

    ebase = wid * EPT

    def chunk_body(i, carry):
        base = pl.multiple_of(ebase + i * CHUNK, 8)
        base_e = pl.multiple_of(jnp.minimum(base, E - CHUNK), 8)
        pltpu.sync_copy(src_hbm.at[pl.ds(base, CHUNK)], idx_s)
        pltpu.sync_copy(dst_hbm.at[pl.ds(base, CHUNK)], idx_d)
        pltpu.sync_copy(ea_hbm.at[pl.ds(base_e, CHUNK)], ea)
        pltpu.async_copy(x_hbm.at[idx_s], rows, sem).wait()
        _relu_add_chunk(rows, ea, CHUNK)
        pltpu.sync_copy(rows, acc.at[idx_d], add=True)
        return carry

    lax.fori_loop(0, NCHUNKS, chunk_body, 0)

    plsc.subcore_barrier()

    @pl.when(cid == 0)
    def _():
        pltpu.sync_copy(acc.at[pl.ds(r0, ROWS_PER_TILE)],
                        out0_hbm.at[pl.ds(r0, ROWS_PER_TILE)])

    @pl.when(cid == 1)
    def _():
        pltpu.sync_copy(acc.at[pl.ds(r0, ROWS_PER_TILE)],
                        out1_hbm.at[pl.ds(r0, ROWS_PER_TILE)])


def _mlp_body(relu_mid, p0_ref, p1_ref, x_ref, w1_ref, b1_ref, w2_ref,
              b2_ref, o_ref):
    x = x_ref[...]
    out = p0_ref[...] + p1_ref[...] + x
    h = jnp.maximum(
        jnp.dot(out, w1_ref[...], preferred_element_type=jnp.float32)
        + b1_ref[...], 0.0)
    y = jnp.dot(h, w2_ref[...], preferred_element_type=jnp.float32) + b2_ref[...]
    if relu_mid:
        y = jnp.maximum(y, 0.0)
    o_ref[...] = y + x


def _mlp(part0, part1, x, W1i, b1i, W2i, b2i, relu_mid):
    R = 1024
    row_spec = pl.BlockSpec((R, D), lambda i: (i, 0))
    full2 = pl.BlockSpec((D, D), lambda i: (0, 0))
    bias = pl.BlockSpec((1, D), lambda i: (0, 0))
    return pl.pallas_call(
        functools.partial(_mlp_body, relu_mid),
        grid=(NP // R,),
        in_specs=[row_spec, row_spec, row_spec, full2, bias, full2, bias],
        out_specs=row_spec,
        out_shape=jax.ShapeDtypeStruct((NP, D), jnp.float32),
    )(part0, part1, x, W1i, b1i.reshape(1, D), W2i, b2i.reshape(1, D))


def kernel(z, edge_index, edge_attr, emb, W1, b1, W2, b2):
    z_pad = jnp.concatenate(
        [z.astype(jnp.int32), jnp.zeros((NP - N,), jnp.int32)])
    src = edge_index[0].astype(jnp.int32)
    dst = edge_index[1].astype(jnp.int32)
    pad_e = E_PAD - E
    src_pad = jnp.concatenate([src, jnp.zeros((pad_e,), jnp.int32)])
    dst_pad = jnp.concatenate([dst, jnp.full((pad_e,), N, jnp.int32)])
    zeros_blk = jnp.zeros((CHUNK, D), jnp.float32)

    x = _embed_kernel()(emb, z_pad)
    for i in range(NUM_CONVS):
        part0, part1 = _conv_kernel()(x, src_pad, dst_pad, edge_attr, zeros_blk)
        x = _mlp(part0, part1, x, W1[i], b1[i], W2[i], b2[i],
                 relu_mid=(i < NUM_CONVS - 1))
    return x[:N]


# speedup vs baseline: 2.0935x; 1.1699x over previous
import functools

import jax
import jax.numpy as jnp
from jax import lax
from jax.experimental import pallas as pl
from jax.experimental.pallas import tpu as pltpu
from jax.experimental.pallas import tpu_sc as plsc

N = 10000
E = 320000
D = 128
HD = D // 2
NUM_CONVS = 3

NC = 2
NS = 16
NW = NC * NS

NP = 10240
CHUNK = 128
NCHUNKS = 80
EPT = NCHUNKS * CHUNK
E_PAD = NW * EPT
ROWS_PER_TILE = NP // NS
EMB_ROWS_PER_TILE = NP // NW
EMB_CHUNK = 80


@functools.cache
def _mesh():
    return plsc.VectorSubcoreMesh(
        core_axis_name="c", subcore_axis_name="s",
        num_cores=NC, num_subcores=NS)


@functools.cache
def _embed_kernel():
    return pl.kernel(
        _embed_body,
        out_type=jax.ShapeDtypeStruct((NP, D), jnp.float32),
        mesh=_mesh(),
        scratch_types=[
            pltpu.VMEM((EMB_CHUNK,), jnp.int32),
            pltpu.VMEM((EMB_CHUNK, D), jnp.float32),
            pltpu.SemaphoreType.DMA,
        ],
    )


def _embed_body(emb_hbm, z_hbm, out_hbm, idx_v, rows_v, sem):
    cid = lax.axis_index("c")
    sid = lax.axis_index("s")
    wid = sid * NC + cid
    base = pl.multiple_of(wid * EMB_ROWS_PER_TILE, 8)
    for i in range(EMB_ROWS_PER_TILE // EMB_CHUNK):
        b = pl.multiple_of(base + i * EMB_CHUNK, 8)
        pltpu.sync_copy(z_hbm.at[pl.ds(b, EMB_CHUNK)], idx_v)
        pltpu.async_copy(emb_hbm.at[idx_v], rows_v, sem).wait()
        pltpu.sync_copy(rows_v, out_hbm.at[pl.ds(b, EMB_CHUNK)])


@functools.cache
def _conv_kernel():
    return pl.kernel(
        _conv_body,
        out_type=[jax.ShapeDtypeStruct((NP, HD), jnp.float32),
                  jax.ShapeDtypeStruct((NP, HD), jnp.float32),
                  jax.ShapeDtypeStruct((NP, HD), jnp.float32),
                  jax.ShapeDtypeStruct((NP, HD), jnp.float32)],
        mesh=_mesh(),
        compiler_params=pltpu.CompilerParams(use_tc_tiling_on_sc=False),
        scratch_types=[
            pltpu.VMEM((NCHUNKS, CHUNK), jnp.int32),
            pltpu.VMEM((NCHUNKS, CHUNK), jnp.int32),
            pltpu.VMEM((CHUNK, HD), jnp.float32),
            pltpu.VMEM((CHUNK, HD), jnp.float32),
            pltpu.VMEM((CHUNK, HD), jnp.float32),
            pltpu.VMEM((CHUNK, HD), jnp.float32),
            pltpu.VMEM((CHUNK, HD), jnp.float32),
            pltpu.VMEM((CHUNK, HD), jnp.float32),
            pltpu.VMEM_SHARED((NP, HD), jnp.float32),
            pltpu.SemaphoreType.DMA,
            pltpu.SemaphoreType.DMA,
            pltpu.SemaphoreType.DMA,
            pltpu.SemaphoreType.DMA,
            pltpu.SemaphoreType.DMA,
            pltpu.SemaphoreType.DMA,
        ],
    )


def _conv_body(xa_hbm, xb_hbm, src_hbm, dst_hbm, eaa_hbm, eab_hbm,
               zeros_hbm,
               out0a_hbm, out0b_hbm, out1a_hbm, out1b_hbm,
               idx_s, idx_d, rin0, rin1, msg0, msg1, eab0, eab1,
               acc, sg0, sg1, se0, se1, ss0, ss1):
    cid = lax.axis_index("c")
    sid = lax.axis_index("s")
    wid = sid * NC + cid
    rin = (rin0, rin1)
    msg = (msg0, msg1)
    eab = (eab0, eab1)
    sg = (sg0, sg1)
    se = (se0, se1)
    ss = (ss0, ss1)

    pltpu.sync_copy(src_hbm.at[wid], idx_s)
    pltpu.sync_copy(dst_hbm.at[wid], idx_d)

    r0 = pl.multiple_of(sid * ROWS_PER_TILE, 8)
    ebase = wid * EPT

    def ea_base(i):
        return pl.multiple_of(
            jnp.minimum(ebase + i * CHUNK, E - CHUNK), 8)

    for xt_hbm, eat_hbm, o0_hbm, o1_hbm in (
            (xa_hbm, eaa_hbm, out0a_hbm, out1a_hbm),
            (xb_hbm, eab_hbm, out0b_hbm, out1b_hbm)):
        for j in range(ROWS_PER_TILE // CHUNK):
            pltpu.sync_copy(zeros_hbm, acc.at[pl.ds(r0 + j * CHUNK, CHUNK)])
        plsc.subcore_barrier()

        for b in range(2):
            pltpu.async_copy(xt_hbm.at[idx_s.at[b]], rin[b], sg[b])
            pltpu.async_copy(eat_hbm.at[pl.ds(ea_base(b), CHUNK)],
                             eab[b], se[b])

        def step(i, b):
            @pl.when(i >= 2)
            def _():
                pltpu.make_async_copy(msg[b], acc.at[idx_d.at[0]],
                                      ss[b]).wait()

            pltpu.make_async_copy(xt_hbm.at[idx_s.at[0]], rin[b],
                                  sg[b]).wait()
            pltpu.make_async_copy(eat_hbm.at[pl.ds(0, CHUNK)],
                                  eab[b], se[b]).wait()

            def crow(r, c):
                for j in range(HD // 16):
                    sl = pl.ds(j * 16, 16)
                    msg[b][r, sl] = jnp.maximum(
                        rin[b][r, sl] + eab[b][r, sl], 0.0)
                return c
            lax.fori_loop(0, CHUNK, crow, 0, unroll=4)

            pltpu.async_copy(msg[b], acc.at[idx_d.at[i]], ss[b], add=True)

            @pl.when(i + 2 < NCHUNKS)
            def _():
                pltpu.async_copy(xt_hbm.at[idx_s.at[i + 2]], rin[b], sg[b])
                pltpu.async_copy(eat_hbm.at[pl.ds(ea_base(i + 2), CHUNK)],
                                 eab[b], se[b])

        def pair(k, c):
            step(2 * k, 0)
            step(2 * k + 1, 1)
            return c
        lax.fori_loop(0, NCHUNKS // 2, pair, 0)

        for b in range(2):
            pltpu.make_async_copy(msg[b], acc.at[idx_d.at[0]], ss[b]).wait()

        plsc.subcore_barrier()

        @pl.when(cid == 0)
        def _():
            pltpu.sync_copy(acc.at[pl.ds(r0, ROWS_PER_TILE)],
                            o0_hbm.at[pl.ds(r0, ROWS_PER_TILE)])

        @pl.when(cid == 1)
        def _():
            pltpu.sync_copy(acc.at[pl.ds(r0, ROWS_PER_TILE)],
                            o1_hbm.at[pl.ds(r0, ROWS_PER_TILE)])
        plsc.subcore_barrier()


def _mlp_body(relu_mid, p0a_ref, p0b_ref, p1a_ref, p1b_ref, x_ref, w1_ref,
              b1_ref, w2_ref, b2_ref, o_ref):
    x = x_ref[...]
    agg = jnp.concatenate([p0a_ref[...] + p1a_ref[...],
                           p0b_ref[...] + p1b_ref[...]], axis=1)
    out = agg + x
    h = jnp.maximum(
        jnp.dot(out, w1_ref[...], preferred_element_type=jnp.float32)
        + b1_ref[...], 0.0)
    y = jnp.dot(h, w2_ref[...], preferred_element_type=jnp.float32) + b2_ref[...]
    if relu_mid:
        y = jnp.maximum(y, 0.0)
    o_ref[...] = y + x


def _mlp(parts, x, W1i, b1i, W2i, b2i, relu_mid):
    R = 1024
    row_spec = pl.BlockSpec((R, D), lambda i: (i, 0))
    half_spec = pl.BlockSpec((R, HD), lambda i: (i, 0))
    full2 = pl.BlockSpec((D, D), lambda i: (0, 0))
    bias = pl.BlockSpec((1, D), lambda i: (0, 0))
    return pl.pallas_call(
        functools.partial(_mlp_body, relu_mid),
        grid=(NP // R,),
        in_specs=[half_spec, half_spec, half_spec, half_spec, row_spec,
                  full2, bias, full2, bias],
        out_specs=row_spec,
        out_shape=jax.ShapeDtypeStruct((NP, D), jnp.float32),
    )(parts[0], parts[1], parts[2], parts[3], x,
      W1i, b1i.reshape(1, D), W2i, b2i.reshape(1, D))


def kernel(z, edge_index, edge_attr, emb, W1, b1, W2, b2):
    z_pad = jnp.concatenate(
        [z.astype(jnp.int32), jnp.zeros((NP - N,), jnp.int32)])
    src = edge_index[0].astype(jnp.int32)
    dst = edge_index[1].astype(jnp.int32)
    pad_e = E_PAD - E
    src_pad = jnp.concatenate(
        [src, jnp.zeros((pad_e,), jnp.int32)]).reshape(NW, NCHUNKS, CHUNK)
    dst_pad = jnp.concatenate(
        [dst, jnp.full((pad_e,), N, jnp.int32)]).reshape(NW, NCHUNKS, CHUNK)
    zeros_blk = jnp.zeros((CHUNK, HD), jnp.float32)

    ea_a = edge_attr[:, :HD]
    ea_b = edge_attr[:, HD:]
    x = _embed_kernel()(emb, z_pad)
    for i in range(NUM_CONVS):
        xa = x[:, :HD]
        xb = x[:, HD:]
        parts = _conv_kernel()(xa, xb, src_pad, dst_pad, ea_a, ea_b,
                               zeros_blk)
        x = _mlp(parts, x, W1[i], b1[i], W2[i], b2[i],
                 relu_mid=(i < NUM_CONVS - 1))
    return x[:N]
